# baseline (speedup 1.0000x reference)
import jax
import jax.numpy as jnp
from jax.experimental import pallas as pl
from jax.experimental.pallas import tpu as pltpu

UNITS = 32
COORD = 4
D_IN = 64
TOTAL = 32768
B = 16
BLK = 4096
NBLK = TOTAL // BLK


def _grc_kernel(rs_ref, node_hbm, coordT_hbm, wr_ref, br_ref, out_ref,
                nbuf, cbuf, macc_ref, csum_ref, nsem, csem):
    ccopy = pltpu.make_async_copy(coordT_hbm, cbuf, csem)
    ccopy.start()
    for i in range(NBLK):
        pltpu.make_async_copy(
            node_hbm.at[pl.ds(i * BLK, BLK), :], nbuf.at[i],
            nsem.at[i]).start()

    macc_ref[...] = jnp.zeros_like(macc_ref)
    csum_ref[...] = jnp.zeros_like(csum_ref)
    ccopy.wait()

    for i in range(NBLK):
        pltpu.make_async_copy(
            node_hbm.at[pl.ds(i * BLK, BLK), :], nbuf.at[i],
            nsem.at[i]).wait()

        idx = i * BLK + jax.lax.broadcasted_iota(jnp.int32, (1, BLK), 1)
        seg = jnp.zeros((1, BLK), jnp.int32)
        for j in range(1, B):
            seg += (idx >= rs_ref[j]).astype(jnp.int32)

        coordT = cbuf[:, i * BLK:(i + 1) * BLK]
        node = nbuf[i]

        rowb = jax.lax.broadcasted_iota(jnp.int32, (B, BLK), 0)
        onehot = jnp.where(rowb == seg, 1.0, 0.0)
        e_t = jnp.concatenate(
            [onehot * coordT[c:c + 1, :] for c in range(COORD)],
            axis=0)

        macc_ref[...] += jax.lax.dot_general(
            e_t, node, (((1,), (0,)), ((), ())),
            preferred_element_type=jnp.float32)
        csum_ref[...] += jax.lax.dot_general(
            e_t, jnp.ones((BLK, 8), jnp.float32), (((1,), (0,)), ((), ())),
            preferred_element_type=jnp.float32)

    acc = jnp.zeros((B, UNITS), jnp.float32)
    for c in range(COORD):
        acc += jnp.dot(macc_ref[c * B:(c + 1) * B, :],
                       wr_ref[c * D_IN:(c + 1) * D_IN, :],
                       preferred_element_type=jnp.float32)
        acc += csum_ref[c * B:(c + 1) * B, 0:1] * br_ref[c:c + 1, :]
    out_ref[...] = acc


def kernel(node_features, coord_features, row_splits, W, b):
    wr = W.reshape(D_IN, UNITS, COORD).transpose(2, 0, 1).reshape(
        COORD * D_IN, UNITS)
    br = b.reshape(UNITS, COORD).T
    coordT = coord_features.T

    return pl.pallas_call(
        _grc_kernel,
        in_specs=[
            pl.BlockSpec(memory_space=pltpu.SMEM),
            pl.BlockSpec(memory_space=pl.ANY),
            pl.BlockSpec(memory_space=pl.ANY),
            pl.BlockSpec((COORD * D_IN, UNITS), lambda: (0, 0)),
            pl.BlockSpec((COORD, UNITS), lambda: (0, 0)),
        ],
        out_specs=pl.BlockSpec((B, UNITS), lambda: (0, 0)),
        out_shape=jax.ShapeDtypeStruct((B, UNITS), jnp.float32),
        scratch_shapes=[
            pltpu.VMEM((NBLK, BLK, D_IN), jnp.float32),
            pltpu.VMEM((COORD, TOTAL), jnp.float32),
            pltpu.VMEM((COORD * B, D_IN), jnp.float32),
            pltpu.VMEM((COORD * B, 8), jnp.float32),
            pltpu.SemaphoreType.DMA((NBLK,)),
            pltpu.SemaphoreType.DMA,
        ],
    )(row_splits, node_features, coordT, wr, br)


# speedup vs baseline: 1.1455x; 1.1455x over previous
import jax
import jax.numpy as jnp
from jax.experimental import pallas as pl
from jax.experimental.pallas import tpu as pltpu

UNITS = 32
COORD = 4
D_IN = 64
TOTAL = 32768
B = 16
BLK = 8192
NBLK = TOTAL // BLK


def _grc_kernel(rs_ref, node_ref, coordT_ref, wr_ref, br_ref, out_ref,
                macc_ref, csum_ref):
    step = pl.program_id(0)

    @pl.when(step == 0)
    def _init():
        macc_ref[...] = jnp.zeros_like(macc_ref)
        csum_ref[...] = jnp.zeros_like(csum_ref)

    idx = step * BLK + jax.lax.broadcasted_iota(jnp.int32, (1, BLK), 1)
    seg = jnp.zeros((1, BLK), jnp.int32)
    for j in range(1, B):
        seg += (idx >= rs_ref[j]).astype(jnp.int32)

    node = node_ref[...]
    coordT = coordT_ref[...]

    rowj = jax.lax.broadcasted_iota(jnp.int32, (COORD * B, BLK), 0)
    rowb = rowj % B
    rowc = rowj // B
    coordsel = jnp.zeros((COORD * B, BLK), jnp.float32)
    for cc in range(COORD):
        coordsel += jnp.where(rowc == cc, coordT[cc:cc + 1, :], 0.0)
    e_t = jnp.where(rowb == seg, coordsel, 0.0)

    macc_ref[...] += jax.lax.dot_general(
        e_t, node, (((1,), (0,)), ((), ())),
        preferred_element_type=jnp.float32)
    csum_ref[...] += jnp.sum(e_t, axis=1, keepdims=True)

    @pl.when(step == NBLK - 1)
    def _epilogue():
        acc = jnp.zeros((B, UNITS), jnp.float32)
        for c in range(COORD):
            acc += jnp.dot(macc_ref[c * B:(c + 1) * B, :],
                           wr_ref[c * D_IN:(c + 1) * D_IN, :],
                           preferred_element_type=jnp.float32)
            acc += csum_ref[c * B:(c + 1) * B, :] * br_ref[c:c + 1, :]
        out_ref[...] = acc


def kernel(node_features, coord_features, row_splits, W, b):
    wr = W.reshape(D_IN, UNITS, COORD).transpose(2, 0, 1).reshape(
        COORD * D_IN, UNITS)
    br = b.reshape(UNITS, COORD).T
    coordT = coord_features.T

    return pl.pallas_call(
        _grc_kernel,
        grid=(NBLK,),
        in_specs=[
            pl.BlockSpec(memory_space=pltpu.SMEM),
            pl.BlockSpec((BLK, D_IN), lambda i: (i, 0)),
            pl.BlockSpec((COORD, BLK), lambda i: (0, i)),
            pl.BlockSpec((COORD * D_IN, UNITS), lambda i: (0, 0)),
            pl.BlockSpec((COORD, UNITS), lambda i: (0, 0)),
        ],
        out_specs=pl.BlockSpec((B, UNITS), lambda i: (0, 0)),
        out_shape=jax.ShapeDtypeStruct((B, UNITS), jnp.float32),
        scratch_shapes=[
            pltpu.VMEM((COORD * B, D_IN), jnp.float32),
            pltpu.VMEM((COORD * B, 1), jnp.float32),
        ],
    )(row_splits, node_features, coordT, wr, br)


# speedup vs baseline: 1.1564x; 1.0095x over previous
import jax
import jax.numpy as jnp
from jax.experimental import pallas as pl
from jax.experimental.pallas import tpu as pltpu

UNITS = 32
COORD = 4
D_IN = 64
TOTAL = 32768
B = 16
BLK = 16384
NBLK = TOTAL // BLK


def _grc_kernel(rs_ref, node_ref, coordT_ref, wr_ref, br_ref, out_ref,
                macc_ref, csum_ref):
    step = pl.program_id(0)

    @pl.when(step == 0)
    def _init():
        macc_ref[...] = jnp.zeros_like(macc_ref)
        csum_ref[...] = jnp.zeros_like(csum_ref)

    idx = step * BLK + jax.lax.broadcasted_iota(jnp.int32, (1, BLK), 1)
    seg = jnp.zeros((1, BLK), jnp.int32)
    for j in range(1, B):
        seg += (idx >= rs_ref[j]).astype(jnp.int32)

    node = node_ref[...]
    coordT = coordT_ref[...]

    rowj = jax.lax.broadcasted_iota(jnp.int32, (COORD * B, BLK), 0)
    rowb = rowj % B
    rowc = rowj // B
    coordsel = jnp.zeros((COORD * B, BLK), jnp.float32)
    for cc in range(COORD):
        coordsel += jnp.where(rowc == cc, coordT[cc:cc + 1, :], 0.0)
    e_t = jnp.where(rowb == seg, coordsel, 0.0)

    macc_ref[...] += jax.lax.dot_general(
        e_t, node, (((1,), (0,)), ((), ())),
        preferred_element_type=jnp.float32)
    csum_ref[...] += jnp.sum(e_t, axis=1, keepdims=True)

    @pl.when(step == NBLK - 1)
    def _epilogue():
        acc = jnp.zeros((B, UNITS), jnp.float32)
        for c in range(COORD):
            acc += jnp.dot(macc_ref[c * B:(c + 1) * B, :],
                           wr_ref[c * D_IN:(c + 1) * D_IN, :],
                           preferred_element_type=jnp.float32)
            acc += csum_ref[c * B:(c + 1) * B, :] * br_ref[c:c + 1, :]
        out_ref[...] = acc


def kernel(node_features, coord_features, row_splits, W, b):
    wr = W.reshape(D_IN, UNITS, COORD).transpose(2, 0, 1).reshape(
        COORD * D_IN, UNITS)
    br = b.reshape(UNITS, COORD).T
    coordT = coord_features.T

    return pl.pallas_call(
        _grc_kernel,
        grid=(NBLK,),
        in_specs=[
            pl.BlockSpec(memory_space=pltpu.SMEM),
            pl.BlockSpec((BLK, D_IN), lambda i: (i, 0)),
            pl.BlockSpec((COORD, BLK), lambda i: (0, i)),
            pl.BlockSpec((COORD * D_IN, UNITS), lambda i: (0, 0)),
            pl.BlockSpec((COORD, UNITS), lambda i: (0, 0)),
        ],
        out_specs=pl.BlockSpec((B, UNITS), lambda i: (0, 0)),
        out_shape=jax.ShapeDtypeStruct((B, UNITS), jnp.float32),
        scratch_shapes=[
            pltpu.VMEM((COORD * B, D_IN), jnp.float32),
            pltpu.VMEM((COORD * B, 1), jnp.float32),
        ],
    )(row_splits, node_features, coordT, wr, br)
